# baseline (speedup 1.0000x reference)
import functools

import jax
import jax.numpy as jnp
from jax import lax
from jax.experimental import pallas as pl
from jax.experimental.pallas import tpu as pltpu

OBS_DIM = 512
ACTION_DIM = 64
N_AGENTS = 8
HIDDEN_DIM = 1024
ASP_DIM = 256
ASP_HIDDEN = 512

BM = 256
OUT_COLS = 80
NCHUNK = 4


def _fused_body(ba_ref, x_ref, W1_ref, b1_ref, W2_ref, b2_ref,
                Wa1_ref, ba1_ref, Wa2_ref, ba2_ref,
                Wv_ref, bv_ref, Wp1_ref, bp1_ref, Wp2_ref, bp2_ref,
                out_ref):
    f32 = jnp.float32
    x = x_ref[...]
    h1 = jnp.maximum(jnp.dot(x, W1_ref[...], preferred_element_type=f32) + b1_ref[...], 0.0)
    h = jnp.maximum(jnp.dot(h1, W2_ref[...], preferred_element_type=f32) + b2_ref[...], 0.0)
    a1 = jnp.maximum(jnp.dot(h, Wa1_ref[0], preferred_element_type=f32) + ba1_ref[0], 0.0)
    f = jnp.dot(a1, Wa2_ref[0], preferred_element_type=f32) + ba2_ref[0]
    p1 = jnp.maximum(
        jnp.dot(h, Wp1_ref[:HIDDEN_DIM, :], preferred_element_type=f32)
        + jnp.dot(f, Wp1_ref[HIDDEN_DIM:, :], preferred_element_type=f32)
        + bp1_ref[...], 0.0)
    logits = jnp.dot(p1, Wp2_ref[...], preferred_element_type=f32) + bp2_ref[...]
    value = (jnp.sum(h * Wv_ref[:, :HIDDEN_DIM], axis=1, keepdims=True)
             + jnp.sum(f * Wv_ref[:, HIDDEN_DIM:], axis=1, keepdims=True)
             + bv_ref[0])
    out_ref[...] = jnp.concatenate(
        [logits, jnp.broadcast_to(value, (value.shape[0], OUT_COLS - ACTION_DIM))], axis=1)


def _fused_net(x_sorted, block_agent, W1, b1, W2, b2, Wa1, ba1, Wa2, ba2,
               Wv, bv, Wp1, bp1, Wp2, bp2, *, interpret=False):
    m_pad = x_sorted.shape[0]
    nb = m_pad // BM
    grid_spec = pltpu.PrefetchScalarGridSpec(
        num_scalar_prefetch=1,
        grid=(nb,),
        in_specs=[
            pl.BlockSpec((BM, OBS_DIM), lambda i, ba: (i, 0)),
            pl.BlockSpec((OBS_DIM, HIDDEN_DIM), lambda i, ba: (0, 0)),
            pl.BlockSpec((1, HIDDEN_DIM), lambda i, ba: (0, 0)),
            pl.BlockSpec((HIDDEN_DIM, HIDDEN_DIM), lambda i, ba: (0, 0)),
            pl.BlockSpec((1, HIDDEN_DIM), lambda i, ba: (0, 0)),
            pl.BlockSpec((1, HIDDEN_DIM, ASP_HIDDEN), lambda i, ba: (ba[i], 0, 0)),
            pl.BlockSpec((1, 1, ASP_HIDDEN), lambda i, ba: (ba[i], 0, 0)),
            pl.BlockSpec((1, ASP_HIDDEN, ASP_DIM), lambda i, ba: (ba[i], 0, 0)),
            pl.BlockSpec((1, 1, ASP_DIM), lambda i, ba: (ba[i], 0, 0)),
            pl.BlockSpec((1, HIDDEN_DIM + ASP_DIM), lambda i, ba: (0, 0)),
            pl.BlockSpec(memory_space=pltpu.SMEM),
            pl.BlockSpec((HIDDEN_DIM + ASP_DIM, HIDDEN_DIM), lambda i, ba: (0, 0)),
            pl.BlockSpec((1, HIDDEN_DIM), lambda i, ba: (0, 0)),
            pl.BlockSpec((HIDDEN_DIM, ACTION_DIM), lambda i, ba: (0, 0)),
            pl.BlockSpec((1, ACTION_DIM), lambda i, ba: (0, 0)),
        ],
        out_specs=pl.BlockSpec((BM, OUT_COLS), lambda i, ba: (i, 0)),
    )
    return pl.pallas_call(
        _fused_body,
        grid_spec=grid_spec,
        out_shape=jax.ShapeDtypeStruct((m_pad, OUT_COLS), jnp.float32),
        interpret=interpret,
    )(block_agent, x_sorted,
      W1, b1.reshape(1, -1), W2, b2.reshape(1, -1),
      Wa1, ba1.reshape(N_AGENTS, 1, ASP_HIDDEN), Wa2, ba2.reshape(N_AGENTS, 1, ASP_DIM),
      Wv.reshape(1, -1), bv, Wp1, bp1.reshape(1, -1), Wp2, bp2.reshape(1, -1))


def _routing(ids, m_pad):
    m = ids.shape[0]
    onehot = (ids[:, None] == jnp.arange(N_AGENTS, dtype=ids.dtype)[None, :]).astype(jnp.int32)
    cum = jnp.cumsum(onehot, axis=0)
    rank = jnp.take_along_axis(cum, ids[:, None].astype(jnp.int32), axis=1)[:, 0] - 1
    counts = cum[-1]
    padded = ((counts + BM - 1) // BM) * BM
    ends = jnp.cumsum(padded)
    offs = ends - padded
    dest = offs[ids] + rank
    tok_at = jnp.zeros((m_pad,), jnp.int32).at[dest].set(jnp.arange(m, dtype=jnp.int32))
    nb = m_pad // BM
    block_start = jnp.arange(nb, dtype=jnp.int32) * BM
    block_agent = jnp.minimum(
        jnp.searchsorted(ends, block_start, side='right').astype(jnp.int32), N_AGENTS - 1)
    return tok_at, dest, block_agent


def kernel(obs, agent_ids, W1, b1, W2, b2, Wa1, ba1, Wa2, ba2, Wv, bv, Wp1, bp1, Wp2, bp2):
    b, n, o = obs.shape
    m = b * n
    m_pad = m + N_AGENTS * BM
    x = obs.reshape(m, o)
    ids = agent_ids.reshape(m).astype(jnp.int32)

    tok_at, dest, block_agent = _routing(ids, m_pad)

    rows_c = m_pad // NCHUNK
    nb_c = rows_c // BM
    outbufs = []
    for c in range(NCHUNK):
        x_c = jnp.take(x, lax.dynamic_slice_in_dim(tok_at, c * rows_c, rows_c), axis=0)
        ba_c = lax.dynamic_slice_in_dim(block_agent, c * nb_c, nb_c)
        outbufs.append(_fused_net(x_c, ba_c, W1, b1, W2, b2, Wa1, ba1, Wa2, ba2,
                                  Wv, bv, Wp1, bp1, Wp2, bp2))
    outbuf = jnp.concatenate(outbufs, axis=0) if NCHUNK > 1 else outbufs[0]
    out = jnp.take(outbuf, dest, axis=0)

    values = out[:, ACTION_DIM].reshape(b, n)
    logits = out[:, :ACTION_DIM].reshape(b, n, ACTION_DIM)
    return (values, logits)


# speedup vs baseline: 1.3715x; 1.3715x over previous
import functools

import jax
import jax.numpy as jnp
from jax import lax
from jax.experimental import pallas as pl
from jax.experimental.pallas import tpu as pltpu

OBS_DIM = 512
ACTION_DIM = 64
N_AGENTS = 8
HIDDEN_DIM = 1024
ASP_DIM = 256
ASP_HIDDEN = 512

BM = 256
OUT_COLS = 80
NCHUNK = 4


def _fused_body(ba_ref, x_ref, W1_ref, b1_ref, W2_ref, b2_ref,
                Wa1_ref, ba1_ref, Wa2_ref, ba2_ref,
                Wv_ref, bv_ref, Wp1_ref, bp1_ref, Wp2_ref, bp2_ref,
                out_ref):
    f32 = jnp.float32
    x = x_ref[...]
    h1 = jnp.maximum(jnp.dot(x, W1_ref[...], preferred_element_type=f32) + b1_ref[...], 0.0)
    h = jnp.maximum(jnp.dot(h1, W2_ref[...], preferred_element_type=f32) + b2_ref[...], 0.0)
    a1 = jnp.maximum(jnp.dot(h, Wa1_ref[0], preferred_element_type=f32) + ba1_ref[0], 0.0)
    f = jnp.dot(a1, Wa2_ref[0], preferred_element_type=f32) + ba2_ref[0]
    p1 = jnp.maximum(
        jnp.dot(h, Wp1_ref[:HIDDEN_DIM, :], preferred_element_type=f32)
        + jnp.dot(f, Wp1_ref[HIDDEN_DIM:, :], preferred_element_type=f32)
        + bp1_ref[...], 0.0)
    logits = jnp.dot(p1, Wp2_ref[...], preferred_element_type=f32) + bp2_ref[...]
    value = (jnp.sum(h * Wv_ref[:, :HIDDEN_DIM], axis=1, keepdims=True)
             + jnp.sum(f * Wv_ref[:, HIDDEN_DIM:], axis=1, keepdims=True)
             + bv_ref[0])
    out_ref[...] = jnp.concatenate(
        [logits, jnp.broadcast_to(value, (value.shape[0], OUT_COLS - ACTION_DIM))], axis=1)


def _fused_net(x_sorted, block_agent, W1, b1, W2, b2, Wa1, ba1, Wa2, ba2,
               Wv, bv, Wp1, bp1, Wp2, bp2, *, interpret=False):
    m_pad = x_sorted.shape[0]
    nb = m_pad // BM
    grid_spec = pltpu.PrefetchScalarGridSpec(
        num_scalar_prefetch=1,
        grid=(nb,),
        in_specs=[
            pl.BlockSpec((BM, OBS_DIM), lambda i, ba: (i, 0)),
            pl.BlockSpec((OBS_DIM, HIDDEN_DIM), lambda i, ba: (0, 0)),
            pl.BlockSpec((1, HIDDEN_DIM), lambda i, ba: (0, 0)),
            pl.BlockSpec((HIDDEN_DIM, HIDDEN_DIM), lambda i, ba: (0, 0)),
            pl.BlockSpec((1, HIDDEN_DIM), lambda i, ba: (0, 0)),
            pl.BlockSpec((1, HIDDEN_DIM, ASP_HIDDEN), lambda i, ba: (ba[i], 0, 0)),
            pl.BlockSpec((1, 1, ASP_HIDDEN), lambda i, ba: (ba[i], 0, 0)),
            pl.BlockSpec((1, ASP_HIDDEN, ASP_DIM), lambda i, ba: (ba[i], 0, 0)),
            pl.BlockSpec((1, 1, ASP_DIM), lambda i, ba: (ba[i], 0, 0)),
            pl.BlockSpec((1, HIDDEN_DIM + ASP_DIM), lambda i, ba: (0, 0)),
            pl.BlockSpec(memory_space=pltpu.SMEM),
            pl.BlockSpec((HIDDEN_DIM + ASP_DIM, HIDDEN_DIM), lambda i, ba: (0, 0)),
            pl.BlockSpec((1, HIDDEN_DIM), lambda i, ba: (0, 0)),
            pl.BlockSpec((HIDDEN_DIM, ACTION_DIM), lambda i, ba: (0, 0)),
            pl.BlockSpec((1, ACTION_DIM), lambda i, ba: (0, 0)),
        ],
        out_specs=pl.BlockSpec((BM, OUT_COLS), lambda i, ba: (i, 0)),
    )
    return pl.pallas_call(
        _fused_body,
        grid_spec=grid_spec,
        out_shape=jax.ShapeDtypeStruct((m_pad, OUT_COLS), jnp.float32),
        interpret=interpret,
    )(block_agent, x_sorted,
      W1, b1.reshape(1, -1), W2, b2.reshape(1, -1),
      Wa1, ba1.reshape(N_AGENTS, 1, ASP_HIDDEN), Wa2, ba2.reshape(N_AGENTS, 1, ASP_DIM),
      Wv.reshape(1, -1), bv, Wp1, bp1.reshape(1, -1), Wp2, bp2.reshape(1, -1))


def _routing_body(ids_ref, dest_ref, padded_ref):
    f32 = jnp.float32
    ids = ids_ref[...]
    r_lt_c = (lax.broadcasted_iota(jnp.int32, (128, 128), 0)
              < lax.broadcasted_iota(jnp.int32, (128, 128), 1)).astype(f32)
    masks = []
    prefs = []
    row_counts = []
    for a in range(N_AGENTS):
        m_a = (ids == a).astype(f32)
        p_a = jnp.dot(m_a, r_lt_c, preferred_element_type=f32)
        masks.append(m_a)
        prefs.append(p_a)
        row_counts.append(p_a[:, 127:128] + m_a[:, 127:128])
    cmat = jnp.concatenate(row_counts, axis=1)
    c_lt_r = (lax.broadcasted_iota(jnp.int32, (128, 128), 1)
              < lax.broadcasted_iota(jnp.int32, (128, 128), 0)).astype(f32)
    cex = jnp.dot(c_lt_r, cmat, preferred_element_type=f32)
    tot = cex[127:128, :] + cmat[127:128, :]
    padded = jnp.ceil(tot / BM) * BM
    a_lt_b = (lax.broadcasted_iota(jnp.int32, (N_AGENTS, N_AGENTS), 0)
              < lax.broadcasted_iota(jnp.int32, (N_AGENTS, N_AGENTS), 1)).astype(f32)
    poff = jnp.dot(padded, a_lt_b, preferred_element_type=f32)
    base = cex + poff
    dest = jnp.zeros((128, 128), f32)
    for a in range(N_AGENTS):
        dest = dest + masks[a] * (prefs[a] + base[:, a:a + 1])
    dest_ref[...] = dest.astype(jnp.int32)
    padded_ref[...] = padded.astype(jnp.int32)


def _routing(ids, m_pad, *, interpret=False):
    dest2, padded = pl.pallas_call(
        _routing_body,
        grid=(1,),
        in_specs=[pl.BlockSpec((128, 128), lambda i: (0, 0))],
        out_specs=[pl.BlockSpec((128, 128), lambda i: (0, 0)),
                   pl.BlockSpec((1, N_AGENTS), lambda i: (0, 0))],
        out_shape=[jax.ShapeDtypeStruct((128, 128), jnp.int32),
                   jax.ShapeDtypeStruct((1, N_AGENTS), jnp.int32)],
        interpret=interpret,
    )(ids.reshape(128, 128))
    dest = dest2.reshape(-1)
    ends = jnp.cumsum(padded[0])
    nb = m_pad // BM
    block_start = jnp.arange(nb, dtype=jnp.int32) * BM
    block_agent = jnp.minimum(
        jnp.searchsorted(ends, block_start, side='right').astype(jnp.int32), N_AGENTS - 1)
    return dest, block_agent


def kernel(obs, agent_ids, W1, b1, W2, b2, Wa1, ba1, Wa2, ba2, Wv, bv, Wp1, bp1, Wp2, bp2):
    b, n, o = obs.shape
    m = b * n
    m_pad = m + N_AGENTS * BM
    x = obs.reshape(m, o)
    ids = agent_ids.reshape(m).astype(jnp.int32)

    dest, block_agent = _routing(ids, m_pad)

    x_sorted = jnp.zeros((m_pad, o), x.dtype).at[dest].set(x, mode='drop', unique_indices=True)
    outbuf = _fused_net(x_sorted, block_agent, W1, b1, W2, b2, Wa1, ba1, Wa2, ba2,
                        Wv, bv, Wp1, bp1, Wp2, bp2)
    out = jnp.take(outbuf, dest, axis=0)

    values = out[:, ACTION_DIM].reshape(b, n)
    logits = out[:, :ACTION_DIM].reshape(b, n, ACTION_DIM)
    return (values, logits)


# speedup vs baseline: 1.6662x; 1.2149x over previous
import functools

import jax
import jax.numpy as jnp
from jax import lax
from jax.experimental import pallas as pl
from jax.experimental.pallas import tpu as pltpu
from jax.experimental.pallas import tpu_sc as plsc

OBS_DIM = 512
ACTION_DIM = 64
N_AGENTS = 8
HIDDEN_DIM = 1024
ASP_DIM = 256
ASP_HIDDEN = 512

BM = 256
OUT_COLS = 80
NCHUNK = 4


def _fused_body(ba_ref, x_ref, W1_ref, b1_ref, W2_ref, b2_ref,
                Wa1_ref, ba1_ref, Wa2_ref, ba2_ref,
                Wv_ref, bv_ref, Wp1_ref, bp1_ref, Wp2_ref, bp2_ref,
                out_ref):
    f32 = jnp.float32
    x = x_ref[...]
    h1 = jnp.maximum(jnp.dot(x, W1_ref[...], preferred_element_type=f32) + b1_ref[...], 0.0)
    h = jnp.maximum(jnp.dot(h1, W2_ref[...], preferred_element_type=f32) + b2_ref[...], 0.0)
    a1 = jnp.maximum(jnp.dot(h, Wa1_ref[0], preferred_element_type=f32) + ba1_ref[0], 0.0)
    f = jnp.dot(a1, Wa2_ref[0], preferred_element_type=f32) + ba2_ref[0]
    p1 = jnp.maximum(
        jnp.dot(h, Wp1_ref[:HIDDEN_DIM, :], preferred_element_type=f32)
        + jnp.dot(f, Wp1_ref[HIDDEN_DIM:, :], preferred_element_type=f32)
        + bp1_ref[...], 0.0)
    logits = jnp.dot(p1, Wp2_ref[...], preferred_element_type=f32) + bp2_ref[...]
    val = (jnp.dot(h, Wv_ref[:HIDDEN_DIM, :], preferred_element_type=f32)
           + jnp.dot(f, Wv_ref[HIDDEN_DIM:, :], preferred_element_type=f32)
           + bv_ref[0])
    out_ref[...] = jnp.concatenate([logits, val[:, :OUT_COLS - ACTION_DIM]], axis=1)


def _fused_net(x_sorted, block_agent, W1, b1, W2, b2, Wa1, ba1, Wa2, ba2,
               Wv, bv, Wp1, bp1, Wp2, bp2, *, interpret=False):
    m_pad = x_sorted.shape[0]
    nb = m_pad // BM
    grid_spec = pltpu.PrefetchScalarGridSpec(
        num_scalar_prefetch=1,
        grid=(nb,),
        in_specs=[
            pl.BlockSpec((BM, OBS_DIM), lambda i, ba: (i, 0)),
            pl.BlockSpec((OBS_DIM, HIDDEN_DIM), lambda i, ba: (0, 0)),
            pl.BlockSpec((1, HIDDEN_DIM), lambda i, ba: (0, 0)),
            pl.BlockSpec((HIDDEN_DIM, HIDDEN_DIM), lambda i, ba: (0, 0)),
            pl.BlockSpec((1, HIDDEN_DIM), lambda i, ba: (0, 0)),
            pl.BlockSpec((1, HIDDEN_DIM, ASP_HIDDEN), lambda i, ba: (ba[i], 0, 0)),
            pl.BlockSpec((1, 1, ASP_HIDDEN), lambda i, ba: (ba[i], 0, 0)),
            pl.BlockSpec((1, ASP_HIDDEN, ASP_DIM), lambda i, ba: (ba[i], 0, 0)),
            pl.BlockSpec((1, 1, ASP_DIM), lambda i, ba: (ba[i], 0, 0)),
            pl.BlockSpec((HIDDEN_DIM + ASP_DIM, 128), lambda i, ba: (0, 0)),
            pl.BlockSpec(memory_space=pltpu.SMEM),
            pl.BlockSpec((HIDDEN_DIM + ASP_DIM, HIDDEN_DIM), lambda i, ba: (0, 0)),
            pl.BlockSpec((1, HIDDEN_DIM), lambda i, ba: (0, 0)),
            pl.BlockSpec((HIDDEN_DIM, ACTION_DIM), lambda i, ba: (0, 0)),
            pl.BlockSpec((1, ACTION_DIM), lambda i, ba: (0, 0)),
        ],
        out_specs=pl.BlockSpec((BM, OUT_COLS), lambda i, ba: (i, 0)),
    )
    return pl.pallas_call(
        _fused_body,
        grid_spec=grid_spec,
        out_shape=jax.ShapeDtypeStruct((m_pad, OUT_COLS), jnp.float32),
        interpret=interpret,
    )(block_agent, x_sorted,
      W1, b1.reshape(1, -1), W2, b2.reshape(1, -1),
      Wa1, ba1.reshape(N_AGENTS, 1, ASP_HIDDEN), Wa2, ba2.reshape(N_AGENTS, 1, ASP_DIM),
      jnp.pad(Wv, ((0, 0), (0, 127))), bv, Wp1, bp1.reshape(1, -1), Wp2, bp2.reshape(1, -1))


def _routing_body(ids_ref, dest_ref, padded_ref):
    f32 = jnp.float32
    ids = ids_ref[...]
    r_lt_c = (lax.broadcasted_iota(jnp.int32, (128, 128), 0)
              < lax.broadcasted_iota(jnp.int32, (128, 128), 1)).astype(f32)
    masks = []
    prefs = []
    row_counts = []
    for a in range(N_AGENTS):
        m_a = (ids == a).astype(f32)
        p_a = jnp.dot(m_a, r_lt_c, preferred_element_type=f32)
        masks.append(m_a)
        prefs.append(p_a)
        row_counts.append(p_a[:, 127:128] + m_a[:, 127:128])
    cmat = jnp.concatenate(row_counts, axis=1)
    c_lt_r = (lax.broadcasted_iota(jnp.int32, (128, 128), 1)
              < lax.broadcasted_iota(jnp.int32, (128, 128), 0)).astype(f32)
    cex = jnp.dot(c_lt_r, cmat, preferred_element_type=f32)
    tot = cex[127:128, :] + cmat[127:128, :]
    padded = jnp.ceil(tot / BM) * BM
    a_lt_b = (lax.broadcasted_iota(jnp.int32, (N_AGENTS, N_AGENTS), 0)
              < lax.broadcasted_iota(jnp.int32, (N_AGENTS, N_AGENTS), 1)).astype(f32)
    poff = jnp.dot(padded, a_lt_b, preferred_element_type=f32)
    base = cex + poff
    dest = jnp.zeros((128, 128), f32)
    for a in range(N_AGENTS):
        dest = dest + masks[a] * (prefs[a] + base[:, a:a + 1])
    dest_ref[...] = dest.astype(jnp.int32)
    padded_ref[...] = padded.astype(jnp.int32)


def _routing(ids, m_pad, *, interpret=False):
    dest2, padded = pl.pallas_call(
        _routing_body,
        grid=(1,),
        in_specs=[pl.BlockSpec((128, 128), lambda i: (0, 0))],
        out_specs=[pl.BlockSpec((128, 128), lambda i: (0, 0)),
                   pl.BlockSpec((1, N_AGENTS), lambda i: (0, 0))],
        out_shape=[jax.ShapeDtypeStruct((128, 128), jnp.int32),
                   jax.ShapeDtypeStruct((1, N_AGENTS), jnp.int32)],
        interpret=interpret,
    )(ids.reshape(128, 128))
    dest = dest2.reshape(-1)
    ends = jnp.cumsum(padded[0])
    nb = m_pad // BM
    block_start = jnp.arange(nb, dtype=jnp.int32) * BM
    block_agent = jnp.minimum(
        jnp.sum((block_start[:, None] >= ends[None, :]).astype(jnp.int32), axis=1),
        N_AGENTS - 1)
    return dest, block_agent


NC = 2
NS = 16
NW = NC * NS
SC_CHUNK = 128


def _sc_scatter_rows(x, dest3, m_pad):
    m, d = x.shape
    n_chunk = m // (NW * SC_CHUNK)
    mesh = plsc.VectorSubcoreMesh(core_axis_name="c", subcore_axis_name="s")

    @functools.partial(
        pl.kernel, mesh=mesh,
        out_type=jax.ShapeDtypeStruct((m_pad, d), jnp.float32),
        scratch_types=[
            pltpu.VMEM((SC_CHUNK,), jnp.int32),
            pltpu.VMEM((SC_CHUNK, d), jnp.float32),
            pltpu.SemaphoreType.DMA,
        ],
    )
    def k(x_hbm, dest_hbm, out_hbm, idx_v, rows_v, sem):
        wid = lax.axis_index("s") * NC + lax.axis_index("c")
        for j in range(n_chunk):
            base = (wid * n_chunk + j) * SC_CHUNK
            pltpu.sync_copy(dest_hbm.at[wid, j], idx_v)
            pltpu.sync_copy(x_hbm.at[pl.ds(base, SC_CHUNK)], rows_v)
            pltpu.async_copy(rows_v, out_hbm.at[idx_v], sem).wait()

    return k(x, dest3)


def kernel(obs, agent_ids, W1, b1, W2, b2, Wa1, ba1, Wa2, ba2, Wv, bv, Wp1, bp1, Wp2, bp2):
    b, n, o = obs.shape
    m = b * n
    m_pad = m + N_AGENTS * BM
    x = obs.reshape(m, o)
    ids = agent_ids.reshape(m).astype(jnp.int32)

    dest, block_agent = _routing(ids, m_pad)

    n_chunk = m // (NW * SC_CHUNK)
    x_sorted = _sc_scatter_rows(x, dest.reshape(NW, n_chunk, SC_CHUNK), m_pad)
    outbuf = _fused_net(x_sorted, block_agent, W1, b1, W2, b2, Wa1, ba1, Wa2, ba2,
                        Wv, bv, Wp1, bp1, Wp2, bp2)
    out = jnp.take(outbuf, dest, axis=0)

    values = out[:, ACTION_DIM].reshape(b, n)
    logits = out[:, :ACTION_DIM].reshape(b, n, ACTION_DIM)
    return (values, logits)


# speedup vs baseline: 1.7891x; 1.0737x over previous
import functools

import jax
import jax.numpy as jnp
from jax import lax
from jax.experimental import pallas as pl
from jax.experimental.pallas import tpu as pltpu
from jax.experimental.pallas import tpu_sc as plsc

OBS_DIM = 512
ACTION_DIM = 64
N_AGENTS = 8
HIDDEN_DIM = 1024
ASP_DIM = 256
ASP_HIDDEN = 512

BM = 256
OUT_COLS = 80
NCHUNK = 4


def _fused_body(ba_ref, x_ref, W1_ref, b1_ref, W2_ref, b2_ref,
                Wa1_ref, ba1_ref, Wa2_ref, ba2_ref,
                Wv_ref, bv_ref, Wp1_ref, bp1_ref, Wp2_ref, bp2_ref,
                out_ref):
    f32 = jnp.float32
    x = x_ref[...]
    h1 = jnp.maximum(jnp.dot(x, W1_ref[...], preferred_element_type=f32) + b1_ref[...], 0.0)
    h = jnp.maximum(jnp.dot(h1, W2_ref[...], preferred_element_type=f32) + b2_ref[...], 0.0)
    a1 = jnp.maximum(jnp.dot(h, Wa1_ref[0], preferred_element_type=f32) + ba1_ref[0], 0.0)
    f = jnp.dot(a1, Wa2_ref[0], preferred_element_type=f32) + ba2_ref[0]
    p1 = jnp.maximum(
        jnp.dot(h, Wp1_ref[:HIDDEN_DIM, :], preferred_element_type=f32)
        + jnp.dot(f, Wp1_ref[HIDDEN_DIM:, :], preferred_element_type=f32)
        + bp1_ref[...], 0.0)
    logits = jnp.dot(p1, Wp2_ref[...], preferred_element_type=f32) + bp2_ref[...]
    value = (jnp.sum(h * Wv_ref[:, :HIDDEN_DIM], axis=1, keepdims=True)
             + jnp.sum(f * Wv_ref[:, HIDDEN_DIM:], axis=1, keepdims=True)
             + bv_ref[0])
    out_ref[...] = jnp.concatenate(
        [logits, jnp.broadcast_to(value, (value.shape[0], OUT_COLS - ACTION_DIM))], axis=1)


def _fused_net(x_sorted, block_agent, W1, b1, W2, b2, Wa1, ba1, Wa2, ba2,
               Wv, bv, Wp1, bp1, Wp2, bp2, *, interpret=False):
    m_pad = x_sorted.shape[0]
    nb = m_pad // BM
    grid_spec = pltpu.PrefetchScalarGridSpec(
        num_scalar_prefetch=1,
        grid=(nb,),
        in_specs=[
            pl.BlockSpec((BM, OBS_DIM), lambda i, ba: (i, 0)),
            pl.BlockSpec((OBS_DIM, HIDDEN_DIM), lambda i, ba: (0, 0)),
            pl.BlockSpec((1, HIDDEN_DIM), lambda i, ba: (0, 0)),
            pl.BlockSpec((HIDDEN_DIM, HIDDEN_DIM), lambda i, ba: (0, 0)),
            pl.BlockSpec((1, HIDDEN_DIM), lambda i, ba: (0, 0)),
            pl.BlockSpec((1, HIDDEN_DIM, ASP_HIDDEN), lambda i, ba: (ba[i], 0, 0)),
            pl.BlockSpec((1, 1, ASP_HIDDEN), lambda i, ba: (ba[i], 0, 0)),
            pl.BlockSpec((1, ASP_HIDDEN, ASP_DIM), lambda i, ba: (ba[i], 0, 0)),
            pl.BlockSpec((1, 1, ASP_DIM), lambda i, ba: (ba[i], 0, 0)),
            pl.BlockSpec((1, HIDDEN_DIM + ASP_DIM), lambda i, ba: (0, 0)),
            pl.BlockSpec(memory_space=pltpu.SMEM),
            pl.BlockSpec((HIDDEN_DIM + ASP_DIM, HIDDEN_DIM), lambda i, ba: (0, 0)),
            pl.BlockSpec((1, HIDDEN_DIM), lambda i, ba: (0, 0)),
            pl.BlockSpec((HIDDEN_DIM, ACTION_DIM), lambda i, ba: (0, 0)),
            pl.BlockSpec((1, ACTION_DIM), lambda i, ba: (0, 0)),
        ],
        out_specs=pl.BlockSpec((BM, OUT_COLS), lambda i, ba: (i, 0)),
    )
    return pl.pallas_call(
        _fused_body,
        grid_spec=grid_spec,
        out_shape=jax.ShapeDtypeStruct((m_pad, OUT_COLS), jnp.float32),
        interpret=interpret,
    )(block_agent, x_sorted,
      W1, b1.reshape(1, -1), W2, b2.reshape(1, -1),
      Wa1, ba1.reshape(N_AGENTS, 1, ASP_HIDDEN), Wa2, ba2.reshape(N_AGENTS, 1, ASP_DIM),
      Wv.reshape(1, -1), bv, Wp1, bp1.reshape(1, -1), Wp2, bp2.reshape(1, -1))


def _routing_body(ids_ref, dest_ref, padded_ref):
    f32 = jnp.float32
    ids = ids_ref[...]
    r_lt_c = (lax.broadcasted_iota(jnp.int32, (128, 128), 0)
              < lax.broadcasted_iota(jnp.int32, (128, 128), 1)).astype(f32)
    masks = []
    prefs = []
    row_counts = []
    for a in range(N_AGENTS):
        m_a = (ids == a).astype(f32)
        p_a = jnp.dot(m_a, r_lt_c, preferred_element_type=f32)
        masks.append(m_a)
        prefs.append(p_a)
        row_counts.append(p_a[:, 127:128] + m_a[:, 127:128])
    cmat = jnp.concatenate(row_counts, axis=1)
    c_lt_r = (lax.broadcasted_iota(jnp.int32, (128, 128), 1)
              < lax.broadcasted_iota(jnp.int32, (128, 128), 0)).astype(f32)
    cex = jnp.dot(c_lt_r, cmat, preferred_element_type=f32)
    tot = cex[127:128, :] + cmat[127:128, :]
    padded = jnp.ceil(tot / BM) * BM
    a_lt_b = (lax.broadcasted_iota(jnp.int32, (N_AGENTS, N_AGENTS), 0)
              < lax.broadcasted_iota(jnp.int32, (N_AGENTS, N_AGENTS), 1)).astype(f32)
    poff = jnp.dot(padded, a_lt_b, preferred_element_type=f32)
    base = cex + poff
    dest = jnp.zeros((128, 128), f32)
    for a in range(N_AGENTS):
        dest = dest + masks[a] * (prefs[a] + base[:, a:a + 1])
    dest_ref[...] = dest.astype(jnp.int32)
    padded_ref[...] = padded.astype(jnp.int32)


def _routing(ids, m_pad, *, interpret=False):
    dest2, padded = pl.pallas_call(
        _routing_body,
        grid=(1,),
        in_specs=[pl.BlockSpec((128, 128), lambda i: (0, 0))],
        out_specs=[pl.BlockSpec((128, 128), lambda i: (0, 0)),
                   pl.BlockSpec((1, N_AGENTS), lambda i: (0, 0))],
        out_shape=[jax.ShapeDtypeStruct((128, 128), jnp.int32),
                   jax.ShapeDtypeStruct((1, N_AGENTS), jnp.int32)],
        interpret=interpret,
    )(ids.reshape(128, 128))
    dest = dest2.reshape(-1)
    ends = jnp.cumsum(padded[0])
    nb = m_pad // BM
    block_start = jnp.arange(nb, dtype=jnp.int32) * BM
    block_agent = jnp.minimum(
        jnp.sum((block_start[:, None] >= ends[None, :]).astype(jnp.int32), axis=1),
        N_AGENTS - 1)
    return dest, block_agent


NC = 2
NS = 16
NW = NC * NS
SC_CHUNK = 128


def _sc_scatter_rows(x, dest3, m_pad):
    m, d = x.shape
    n_chunk = m // (NW * SC_CHUNK)
    mesh = plsc.VectorSubcoreMesh(core_axis_name="c", subcore_axis_name="s")

    @functools.partial(
        pl.kernel, mesh=mesh,
        out_type=jax.ShapeDtypeStruct((m_pad, d), jnp.float32),
        scratch_types=[
            pltpu.VMEM((SC_CHUNK,), jnp.int32),
            pltpu.VMEM((SC_CHUNK, d), jnp.float32),
            pltpu.SemaphoreType.DMA,
        ],
    )
    def k(x_hbm, dest_hbm, out_hbm, idx_v, rows_v, sem):
        wid = lax.axis_index("s") * NC + lax.axis_index("c")
        for j in range(n_chunk):
            base = (wid * n_chunk + j) * SC_CHUNK
            pltpu.sync_copy(dest_hbm.at[wid, j], idx_v)
            pltpu.sync_copy(x_hbm.at[pl.ds(base, SC_CHUNK)], rows_v)
            pltpu.async_copy(rows_v, out_hbm.at[idx_v], sem).wait()

    return k(x, dest3)


def kernel(obs, agent_ids, W1, b1, W2, b2, Wa1, ba1, Wa2, ba2, Wv, bv, Wp1, bp1, Wp2, bp2):
    b, n, o = obs.shape
    m = b * n
    m_pad = m + N_AGENTS * BM
    x = obs.reshape(m, o)
    ids = agent_ids.reshape(m).astype(jnp.int32)

    dest, block_agent = _routing(ids, m_pad)

    n_chunk = m // (NW * SC_CHUNK)
    x_sorted = _sc_scatter_rows(x, dest.reshape(NW, n_chunk, SC_CHUNK), m_pad)
    outbuf = _fused_net(x_sorted, block_agent, W1, b1, W2, b2, Wa1, ba1, Wa2, ba2,
                        Wv, bv, Wp1, bp1, Wp2, bp2)
    out = jnp.take(outbuf, dest, axis=0, mode='clip')

    values = out[:, ACTION_DIM].reshape(b, n)
    logits = out[:, :ACTION_DIM].reshape(b, n, ACTION_DIM)
    return (values, logits)
